# baseline (speedup 1.0000x reference)
import functools

import jax
import jax.numpy as jnp
from jax import lax
from jax.experimental import pallas as pl
from jax.experimental.pallas import tpu as pltpu
from jax.experimental.pallas import tpu_sc as plsc

D = 1024
L = 16
NW = 32
B = 4
ROWS = 32768
S_LEN = 8192
S_PER_W = S_LEN // NW
K = 8
NSC = S_PER_W // K
GR = B * K
NG = 3
NPB = 2
NQ = 4
QV = D // L // NQ

_mesh = plsc.VectorSubcoreMesh(core_axis_name="c", subcore_axis_name="s")


@functools.partial(
    pl.kernel,
    mesh=_mesh,
    out_type=jax.ShapeDtypeStruct((ROWS, D), jnp.float32),
    scratch_types=[
        pltpu.VMEM((NSC, GR), jnp.int32),
        pltpu.VMEM((NG, GR, D), jnp.float32),
        pltpu.VMEM((NPB, K, D), jnp.float32),
        pltpu.SemaphoreType.DMA((NG,)),
        pltpu.SemaphoreType.DMA((NPB,)),
        pltpu.SemaphoreType.DMA((NG * B,)),
    ],
)
def _emb_kernel(idx_hbm, tok_hbm, pos_hbm, out_hbm,
                idx_v, rows_v, pos_v, sem_g, sem_p, sem_o):
    cid = lax.axis_index("c")
    sid = lax.axis_index("s")
    wid = sid * 2 + cid
    w_s0 = wid * S_PER_W

    pltpu.sync_copy(idx_hbm.at[pl.ds(wid * NSC, NSC)], idx_v)

    def gather(sc, g):
        pltpu.async_copy(tok_hbm.at[idx_v.at[sc]], rows_v.at[g], sem_g.at[g])

    def wait_gather(sc, g):
        pltpu.make_async_copy(tok_hbm.at[idx_v.at[sc]], rows_v.at[g],
                              sem_g.at[g]).wait()

    def posload(sc, slot):
        pltpu.async_copy(pos_hbm.at[pl.ds(w_s0 + sc * K, K)], pos_v.at[slot],
                         sem_p.at[slot])

    def wait_pos(sc, slot):
        pltpu.make_async_copy(pos_hbm.at[pl.ds(w_s0 + sc * K, K)],
                              pos_v.at[slot], sem_p.at[slot]).wait()

    def outwrite(sc, b, g):
        pltpu.async_copy(rows_v.at[g, pl.ds(b * K, K)],
                         out_hbm.at[pl.ds(b * S_LEN + w_s0 + sc * K, K)],
                         sem_o.at[g * B + b])

    def wait_out(sc, b, g):
        pltpu.make_async_copy(rows_v.at[g, pl.ds(b * K, K)],
                              out_hbm.at[pl.ds(b * S_LEN + w_s0 + sc * K, K)],
                              sem_o.at[g * B + b]).wait()

    gather(0, 0)
    gather(1, 1)
    posload(0, 0)

    def body(sc, carry):
        g = lax.rem(sc, NG)
        gp = lax.rem(sc, NPB)
        gt = lax.rem(sc + 2, NG)

        @pl.when(sc < NSC - 1)
        def _():
            posload(sc + 1, lax.rem(sc + 1, NPB))

        wait_gather(sc, g)
        wait_pos(sc, gp)

        def row_body(i, c2):
            for q in range(NQ):
                pv = [pos_v[gp, i, pl.ds((q * QV + j) * L, L)]
                      for j in range(QV)]
                for b in range(B):
                    for j in range(QV):
                        plsc.addupdate(
                            rows_v.at[g, b * K + i,
                                      pl.ds((q * QV + j) * L, L)],
                            pv[j])
            return c2

        lax.fori_loop(0, K, row_body, 0)

        for b in range(B):
            outwrite(sc, b, g)

        @pl.when((sc >= 1) & (sc < NSC - 2))
        def _():
            for b in range(B):
                wait_out(sc - 1, b, gt)
            gather(sc + 2, gt)

        @pl.when(sc == 0)
        def _():
            gather(2, gt)
        return carry

    lax.fori_loop(0, NSC, body, 0)

    for sc in (NSC - 3, NSC - 2, NSC - 1):
        for b in range(B):
            wait_out(sc, b, sc % NG)


def kernel(x, token_table, pos_table):
    b, s = x.shape
    idx = (x.astype(jnp.int32)
           .reshape(B, NW, NSC, K)
           .transpose(1, 2, 0, 3)
           .reshape(NW * NSC, GR))
    out = _emb_kernel(idx, token_table, pos_table)
    return out.reshape(b, s, D)


# speedup vs baseline: 1.0300x; 1.0300x over previous
import functools

import jax
import jax.numpy as jnp
from jax import lax
from jax.experimental import pallas as pl
from jax.experimental.pallas import tpu as pltpu
from jax.experimental.pallas import tpu_sc as plsc

D = 1024
L = 16
NW = 32
B = 4
ROWS = 32768
S_LEN = 8192
S_PER_W = S_LEN // NW
K = 8
NSC = S_PER_W // K
GR = B * K
NG = 3
NPB = 2
NQ = 4
QV = D // L // NQ

_mesh = plsc.VectorSubcoreMesh(core_axis_name="c", subcore_axis_name="s")


@functools.partial(
    pl.kernel,
    mesh=_mesh,
    out_type=jax.ShapeDtypeStruct((ROWS, D), jnp.float32),
    scratch_types=[
        pltpu.VMEM((NSC, GR), jnp.int32),
        pltpu.VMEM((NG, GR, D), jnp.float32),
        pltpu.VMEM((NPB, K, D), jnp.float32),
        pltpu.SemaphoreType.DMA((NG,)),
        pltpu.SemaphoreType.DMA((NPB,)),
        pltpu.SemaphoreType.DMA((NG * B,)),
    ],
)
def _emb_kernel(idx_hbm, tok_hbm, pos_hbm, out_hbm,
                idx_v, rows_v, pos_v, sem_g, sem_p, sem_o):
    cid = lax.axis_index("c")
    sid = lax.axis_index("s")
    wid = sid * 2 + cid
    w_s0 = wid * S_PER_W

    pltpu.sync_copy(idx_hbm.at[pl.ds(wid * NSC, NSC)], idx_v)

    def gather(sc, g):
        pltpu.async_copy(tok_hbm.at[idx_v.at[sc]], rows_v.at[g], sem_g.at[g])

    def wait_gather(sc, g):
        pltpu.make_async_copy(tok_hbm.at[idx_v.at[sc]], rows_v.at[g],
                              sem_g.at[g]).wait()

    def posload(sc, slot):
        pltpu.async_copy(pos_hbm.at[pl.ds(w_s0 + sc * K, K)], pos_v.at[slot],
                         sem_p.at[slot])

    def wait_pos(sc, slot):
        pltpu.make_async_copy(pos_hbm.at[pl.ds(w_s0 + sc * K, K)],
                              pos_v.at[slot], sem_p.at[slot]).wait()

    def outwrite(sc, b, g):
        pltpu.async_copy(rows_v.at[g, pl.ds(b * K, K)],
                         out_hbm.at[pl.ds(b * S_LEN + w_s0 + sc * K, K)],
                         sem_o.at[g * B + b])

    def wait_out(sc, b, g):
        pltpu.make_async_copy(rows_v.at[g, pl.ds(b * K, K)],
                              out_hbm.at[pl.ds(b * S_LEN + w_s0 + sc * K, K)],
                              sem_o.at[g * B + b]).wait()

    gather(0, 0)
    gather(1, 1)
    posload(0, 0)

    def body(sc, carry):
        g = lax.rem(sc, NG)
        gp = lax.rem(sc, NPB)
        gt = lax.rem(sc + 2, NG)

        @pl.when((sc >= 1) & (sc < NSC - 2))
        def _():
            for b in range(B):
                wait_out(sc - 1, b, gt)
            gather(sc + 2, gt)

        @pl.when(sc == 0)
        def _():
            gather(2, gt)

        @pl.when(sc < NSC - 1)
        def _():
            posload(sc + 1, lax.rem(sc + 1, NPB))

        wait_gather(sc, g)
        wait_pos(sc, gp)

        def row_body(i, c2):
            for q in range(NQ):
                pv = [pos_v[gp, i, pl.ds((q * QV + j) * L, L)]
                      for j in range(QV)]
                for b in range(B):
                    for j in range(QV):
                        plsc.addupdate(
                            rows_v.at[g, b * K + i,
                                      pl.ds((q * QV + j) * L, L)],
                            pv[j])
            return c2

        lax.fori_loop(0, K, row_body, 0)

        for b in range(B):
            outwrite(sc, b, g)
        return carry

    lax.fori_loop(0, NSC, body, 0)

    for sc in (NSC - 3, NSC - 2, NSC - 1):
        for b in range(B):
            wait_out(sc, b, sc % NG)


def kernel(x, token_table, pos_table):
    b, s = x.shape
    idx = (x.astype(jnp.int32)
           .reshape(B, NW, NSC, K)
           .transpose(1, 2, 0, 3)
           .reshape(NW * NSC, GR))
    out = _emb_kernel(idx, token_table, pos_table)
    return out.reshape(b, s, D)
